# baseline (speedup 1.0000x reference)
import functools

import jax
import jax.numpy as jnp
from jax import lax
from jax.experimental import pallas as pl
from jax.experimental.pallas import tpu as pltpu
from jax.experimental.pallas import tpu_sc as plsc

B = 16384
D = 50
CHUNK = 128
PACK = 128
OFF_B = 64
L = 16

NC, NS = 2, 16
NW = NC * NS
BPW = B // NW
NCH = BPW // CHUNK
ROUND = 128
NR = BPW // ROUND

BK = 2048


def _gather_users_body(uidx_h, mfuT_h, ncuT_h, gu_o,
                       idx_u, gu_buf, ring_n0, ring_n1, ring_m0,
                       ring_m1, semu0, semu1):
    wid = lax.axis_index("s") * NC + lax.axis_index("c")
    base = wid * BPW
    pltpu.sync_copy(uidx_h.at[pl.ds(base, BPW)], idx_u.at[pl.ds(0, BPW)])

    rings = ((ring_n0, ring_m0, semu0), (ring_n1, ring_m1, semu1))

    def u_issue(k, slot):
        ring_n, ring_m, semu = rings[slot]
        u = idx_u[pl.ds(k, L)][0]
        c = u // CHUNK
        pltpu.async_copy(ncuT_h.at[:, pl.ds(c * CHUNK, CHUNK)],
                         ring_n.at[pl.ds(0, D)], semu)
        pltpu.async_copy(mfuT_h.at[:, pl.ds(c * CHUNK, CHUNK)],
                         ring_m.at[pl.ds(0, D)], semu)

    def u_extract(k, slot):
        ring_n, ring_m, semu = rings[slot]
        pltpu.make_async_copy(ncuT_h.at[:, pl.ds(0, CHUNK)],
                              ring_n.at[pl.ds(0, D)], semu).wait()
        pltpu.make_async_copy(mfuT_h.at[:, pl.ds(0, CHUNK)],
                              ring_m.at[pl.ds(0, D)], semu).wait()
        u = idx_u[pl.ds(k, L)][0]
        lane = jnp.full((L,), u % CHUNK, dtype=jnp.int32)
        for seg in range(4):
            rows = lax.iota(jnp.int32, L) + seg * L
            vn = plsc.load_gather(ring_n, [rows, lane])
            vm = plsc.load_gather(ring_m, [rows, lane])
            gu_buf[k, pl.ds(seg * L, L)] = vn
            gu_buf[k, pl.ds(OFF_B + seg * L, L)] = vm

    u_issue(0, 0)

    def u_loop(k2, _):
        k = 2 * k2
        u_issue(k + 1, 1)
        u_extract(k, 0)

        @pl.when(k + 2 < BPW)
        def _():
            u_issue(k + 2, 0)
        u_extract(k + 1, 1)
        return ()
    lax.fori_loop(0, BPW // 2, u_loop, ())
    pltpu.sync_copy(gu_buf.at[pl.ds(0, BPW)], gu_o.at[pl.ds(base, BPW)])


def _gather_items_body(uidx_h, iidx_h, mfi_h, nci_h, ubt_h, ibt_h,
                       gi_o, ub_o, ib_o,
                       idx_u, idx_i, gi_buf, ub_v, ib_v, semi):
    wid = lax.axis_index("s") * NC + lax.axis_index("c")
    base = wid * BPW
    pltpu.sync_copy(uidx_h.at[pl.ds(base, BPW)], idx_u.at[pl.ds(0, BPW)])
    pltpu.sync_copy(iidx_h.at[pl.ds(base, BPW)], idx_i.at[pl.ds(0, BPW)])
    for j in range(NCH):
        pltpu.async_copy(ubt_h.at[idx_u.at[pl.ds(j * CHUNK, CHUNK)]],
                         ub_v.at[pl.ds(j * CHUNK, CHUNK)], semi)
        pltpu.async_copy(ibt_h.at[idx_i.at[pl.ds(j * CHUNK, CHUNK)]],
                         ib_v.at[pl.ds(j * CHUNK, CHUNK)], semi)

    for r in range(NR):
        def row_loop(k16, _, r=r):
            kbase = k16 * L
            ivec = idx_i[pl.ds(r * ROUND + kbase, L)]
            for j in range(L):
                i = ivec[j]
                k = kbase + j
                pltpu.async_copy(nci_h.at[i], gi_buf.at[k, pl.ds(0, D)], semi)
                pltpu.async_copy(mfi_h.at[i], gi_buf.at[k, pl.ds(OFF_B, D)],
                                 semi)
            return ()
        lax.fori_loop(0, ROUND // L, row_loop, ())

        def drain_loop(k, _):
            pltpu.make_async_copy(nci_h.at[0], gi_buf.at[0, pl.ds(0, D)],
                                  semi).wait()
            pltpu.make_async_copy(mfi_h.at[0], gi_buf.at[0, pl.ds(OFF_B, D)],
                                  semi).wait()
            return ()
        lax.fori_loop(0, ROUND, drain_loop, (), unroll=4)
        pltpu.sync_copy(gi_buf, gi_o.at[pl.ds(base + r * ROUND, ROUND)])

    for j in range(NCH):
        pltpu.make_async_copy(ubt_h.at[pl.ds(0, CHUNK)],
                              ub_v.at[pl.ds(0, CHUNK)], semi).wait()
        pltpu.make_async_copy(ibt_h.at[pl.ds(0, CHUNK)],
                              ib_v.at[pl.ds(0, CHUNK)], semi).wait()
    pltpu.sync_copy(ub_v, ub_o.at[pl.ds(base, BPW)])
    pltpu.sync_copy(ib_v, ib_o.at[pl.ds(base, BPW)])


_MESH_KW = dict(core_axis_name="c", subcore_axis_name="s",
                num_cores=NC, num_subcores=NS)


@functools.cache
def _build_gather_users():
    return pl.kernel(
        _gather_users_body,
        out_type=[jax.ShapeDtypeStruct((B, PACK), jnp.float32)],
        mesh=plsc.VectorSubcoreMesh(**_MESH_KW),
        compiler_params=pltpu.CompilerParams(needs_layout_passes=False),
        scratch_types=[
            pltpu.VMEM((BPW + L,), jnp.int32),
            pltpu.VMEM((BPW, PACK), jnp.float32),
            pltpu.VMEM((D, CHUNK), jnp.float32),
            pltpu.VMEM((D, CHUNK), jnp.float32),
            pltpu.VMEM((D, CHUNK), jnp.float32),
            pltpu.VMEM((D, CHUNK), jnp.float32),
            pltpu.SemaphoreType.DMA,
            pltpu.SemaphoreType.DMA,
        ],
    )


@functools.cache
def _build_gather_items():
    return pl.kernel(
        _gather_items_body,
        out_type=[
            jax.ShapeDtypeStruct((B, PACK), jnp.float32),
            jax.ShapeDtypeStruct((B,), jnp.float32),
            jax.ShapeDtypeStruct((B,), jnp.float32),
        ],
        mesh=plsc.VectorSubcoreMesh(**_MESH_KW),
        compiler_params=pltpu.CompilerParams(needs_layout_passes=False),
        scratch_types=[
            pltpu.VMEM((BPW + L,), jnp.int32),
            pltpu.VMEM((BPW + L,), jnp.int32),
            pltpu.VMEM((ROUND, PACK), jnp.float32),
            pltpu.VMEM((BPW,), jnp.float32),
            pltpu.VMEM((BPW,), jnp.float32),
            pltpu.SemaphoreType.DMA,
        ],
    )


def _mlp_body(gu, gi, ub, ib, w1u, w1i, b1, w2, b2, w3, b3, wo, bo, fw, fb,
              out):
    guv = gu[...]
    giv = gi[...]
    nue = guv[:, :D]
    ue = guv[:, OFF_B:OFF_B + D]
    nie = giv[:, :D]
    ie = giv[:, OFF_B:OFF_B + D]
    mf = jnp.sum(ue * ie, axis=1) + ub[...] + ib[...]
    h = jnp.dot(nue, w1u[...], preferred_element_type=jnp.float32)
    h += jnp.dot(nie, w1i[...], preferred_element_type=jnp.float32)
    h = jnp.maximum(h + b1[...], 0.0)
    h = jnp.maximum(
        jnp.dot(h, w2[...], preferred_element_type=jnp.float32) + b2[...], 0.0)
    h = jnp.maximum(
        jnp.dot(h, w3[...], preferred_element_type=jnp.float32) + b3[...], 0.0)
    npred = jnp.sum(h * wo[...], axis=1) + bo[0]
    out[...] = mf * fw[0] + npred * fw[1] + fb[0]


def _make_mlp(interpret=False):
    nb = B // BK
    pack_spec = pl.BlockSpec((BK, PACK), lambda i: (i, 0))
    vec_spec = pl.BlockSpec((BK,), lambda i: (i,))

    def full(shape):
        return pl.BlockSpec(shape, lambda i: tuple(0 for _ in shape))

    smem = pl.BlockSpec(memory_space=pltpu.SMEM)
    return pl.pallas_call(
        _mlp_body,
        grid=(nb,),
        in_specs=[
            pack_spec, pack_spec, vec_spec, vec_spec,
            full((D, 100)), full((D, 100)), full((1, 100)),
            full((100, 50)), full((1, 50)),
            full((50, 20)), full((1, 20)),
            full((1, 20)),
            smem, smem, smem,
        ],
        out_specs=vec_spec,
        out_shape=jax.ShapeDtypeStruct((B,), jnp.float32),
        interpret=interpret,
    )


_mlp = _make_mlp()


def kernel(user_indices, item_indices, mf_user_emb, mf_item_emb,
           mf_user_bias, mf_item_bias, ncf_user_emb, ncf_item_emb,
           W1, b1, W2, b2, W3, b3, W_out, b_out, fusion_W, fusion_b):
    uidx = user_indices.astype(jnp.int32)
    iidx = item_indices.astype(jnp.int32)
    gu, = _build_gather_users()(uidx, mf_user_emb.T, ncf_user_emb.T)
    gi, ub, ib = _build_gather_items()(
        uidx, iidx, mf_item_emb, ncf_item_emb, mf_user_bias, mf_item_bias)
    return _mlp(
        gu, gi, ub, ib,
        W1[:D], W1[D:], b1.reshape(1, -1),
        W2, b2.reshape(1, -1), W3, b3.reshape(1, -1),
        W_out.reshape(1, -1), b_out,
        fusion_W.reshape(-1), fusion_b)


# speedup vs baseline: 1.0786x; 1.0786x over previous
import functools

import jax
import jax.numpy as jnp
from jax import lax
from jax.experimental import pallas as pl
from jax.experimental.pallas import tpu as pltpu
from jax.experimental.pallas import tpu_sc as plsc

B = 16384
D = 50
CHUNK = 128
PACK = 128
OFF_B = 64
L = 16

NC, NS = 2, 16
NW = NC * NS
BPW = B // NW
NCH = BPW // CHUNK
ROUND = 128
NR = BPW // ROUND

BK = 2048


def _gather_users_body(uidx_h, mfuT_h, ncuT_h, gu_o,
                       idx_u, gu_buf, ring_n0, ring_n1, ring_m0,
                       ring_m1, semu0, semu1):
    wid = lax.axis_index("s") * NC + lax.axis_index("c")
    base = wid * BPW
    pltpu.sync_copy(uidx_h.at[pl.ds(base, BPW)], idx_u.at[pl.ds(0, BPW)])

    rings = ((ring_n0, ring_m0, semu0), (ring_n1, ring_m1, semu1))

    def u_issue(k, slot):
        ring_n, ring_m, semu = rings[slot]
        u = idx_u[pl.ds(k, L)][0]
        c = u // CHUNK
        pltpu.async_copy(ncuT_h.at[:, pl.ds(c * CHUNK, CHUNK)],
                         ring_n.at[pl.ds(0, D)], semu)
        pltpu.async_copy(mfuT_h.at[:, pl.ds(c * CHUNK, CHUNK)],
                         ring_m.at[pl.ds(0, D)], semu)

    def u_extract(k, slot):
        ring_n, ring_m, semu = rings[slot]
        pltpu.make_async_copy(ncuT_h.at[:, pl.ds(0, CHUNK)],
                              ring_n.at[pl.ds(0, D)], semu).wait()
        pltpu.make_async_copy(mfuT_h.at[:, pl.ds(0, CHUNK)],
                              ring_m.at[pl.ds(0, D)], semu).wait()
        u = idx_u[pl.ds(k, L)][0]
        lane = jnp.full((L,), u % CHUNK, dtype=jnp.int32)
        for seg in range(4):
            rows = lax.iota(jnp.int32, L) + seg * L
            vn = plsc.load_gather(ring_n, [rows, lane])
            vm = plsc.load_gather(ring_m, [rows, lane])
            gu_buf[k, pl.ds(seg * L, L)] = vn
            gu_buf[k, pl.ds(OFF_B + seg * L, L)] = vm

    u_issue(0, 0)

    def u_loop(k2, _):
        k = 2 * k2
        u_issue(k + 1, 1)
        u_extract(k, 0)

        @pl.when(k + 2 < BPW)
        def _():
            u_issue(k + 2, 0)
        u_extract(k + 1, 1)
        return ()
    lax.fori_loop(0, BPW // 2, u_loop, ())
    pltpu.sync_copy(gu_buf.at[pl.ds(0, BPW)], gu_o.at[pl.ds(base, BPW)])


def _gather_items_body(uidx_h, iidx_h, mfi_h, nci_h, ubt_h, ibt_h,
                       gi_o, ub_o, ib_o,
                       idx_u, idx_i, gi_buf, ub_v, ib_v, semi):
    wid = lax.axis_index("s") * NC + lax.axis_index("c")
    base = wid * BPW
    pltpu.sync_copy(uidx_h.at[pl.ds(base, BPW)], idx_u.at[pl.ds(0, BPW)])
    pltpu.sync_copy(iidx_h.at[pl.ds(base, BPW)], idx_i.at[pl.ds(0, BPW)])
    for j in range(NCH):
        pltpu.async_copy(ubt_h.at[idx_u.at[pl.ds(j * CHUNK, CHUNK)]],
                         ub_v.at[pl.ds(j * CHUNK, CHUNK)], semi)
        pltpu.async_copy(ibt_h.at[idx_i.at[pl.ds(j * CHUNK, CHUNK)]],
                         ib_v.at[pl.ds(j * CHUNK, CHUNK)], semi)

    for r in range(NR):
        def row_loop(k16, _, r=r):
            kbase = k16 * L
            ivec = idx_i[pl.ds(r * ROUND + kbase, L)]
            for j in range(L):
                i = ivec[j]
                k = kbase + j
                pltpu.async_copy(nci_h.at[i], gi_buf.at[k, pl.ds(0, D)], semi)
                pltpu.async_copy(mfi_h.at[i], gi_buf.at[k, pl.ds(OFF_B, D)],
                                 semi)
            return ()
        lax.fori_loop(0, ROUND // L, row_loop, ())

        def drain_loop(k, _):
            pltpu.make_async_copy(nci_h.at[0], gi_buf.at[0, pl.ds(0, D)],
                                  semi).wait()
            pltpu.make_async_copy(mfi_h.at[0], gi_buf.at[0, pl.ds(OFF_B, D)],
                                  semi).wait()
            return ()
        lax.fori_loop(0, ROUND, drain_loop, (), unroll=4)
        pltpu.sync_copy(gi_buf, gi_o.at[pl.ds(base + r * ROUND, ROUND)])

    for j in range(NCH):
        pltpu.make_async_copy(ubt_h.at[pl.ds(0, CHUNK)],
                              ub_v.at[pl.ds(0, CHUNK)], semi).wait()
        pltpu.make_async_copy(ibt_h.at[pl.ds(0, CHUNK)],
                              ib_v.at[pl.ds(0, CHUNK)], semi).wait()
    pltpu.sync_copy(ub_v, ub_o.at[pl.ds(base, BPW)])
    pltpu.sync_copy(ib_v, ib_o.at[pl.ds(base, BPW)])


_MESH_KW = dict(core_axis_name="c", subcore_axis_name="s",
                num_cores=NC, num_subcores=NS)


@functools.cache
def _build_gather_users():
    return pl.kernel(
        _gather_users_body,
        out_type=[jax.ShapeDtypeStruct((B, PACK), jnp.float32)],
        mesh=plsc.VectorSubcoreMesh(**_MESH_KW),
        compiler_params=pltpu.CompilerParams(needs_layout_passes=False),
        cost_estimate=pl.CostEstimate(flops=0, transcendentals=0,
                                      bytes_accessed=900_000_000),
        scratch_types=[
            pltpu.VMEM((BPW + L,), jnp.int32),
            pltpu.VMEM((BPW, PACK), jnp.float32),
            pltpu.VMEM((D, CHUNK), jnp.float32),
            pltpu.VMEM((D, CHUNK), jnp.float32),
            pltpu.VMEM((D, CHUNK), jnp.float32),
            pltpu.VMEM((D, CHUNK), jnp.float32),
            pltpu.SemaphoreType.DMA,
            pltpu.SemaphoreType.DMA,
        ],
    )


@functools.cache
def _build_gather_items():
    return pl.kernel(
        _gather_items_body,
        out_type=[
            jax.ShapeDtypeStruct((B, PACK), jnp.float32),
            jax.ShapeDtypeStruct((B,), jnp.float32),
            jax.ShapeDtypeStruct((B,), jnp.float32),
        ],
        mesh=plsc.VectorSubcoreMesh(**_MESH_KW),
        compiler_params=pltpu.CompilerParams(needs_layout_passes=False),
        scratch_types=[
            pltpu.VMEM((BPW + L,), jnp.int32),
            pltpu.VMEM((BPW + L,), jnp.int32),
            pltpu.VMEM((ROUND, PACK), jnp.float32),
            pltpu.VMEM((BPW,), jnp.float32),
            pltpu.VMEM((BPW,), jnp.float32),
            pltpu.SemaphoreType.DMA,
        ],
    )


def _mlp_body(gu, gi, ub, ib, w1u, w1i, b1, w2, b2, w3, b3, wo, bo, fw, fb,
              out):
    guv = gu[...]
    giv = gi[...]
    nue = guv[:, :D]
    ue = guv[:, OFF_B:OFF_B + D]
    nie = giv[:, :D]
    ie = giv[:, OFF_B:OFF_B + D]
    mf = jnp.sum(ue * ie, axis=1) + ub[...] + ib[...]
    h = jnp.dot(nue, w1u[...], preferred_element_type=jnp.float32)
    h += jnp.dot(nie, w1i[...], preferred_element_type=jnp.float32)
    h = jnp.maximum(h + b1[...], 0.0)
    h = jnp.maximum(
        jnp.dot(h, w2[...], preferred_element_type=jnp.float32) + b2[...], 0.0)
    h = jnp.maximum(
        jnp.dot(h, w3[...], preferred_element_type=jnp.float32) + b3[...], 0.0)
    npred = jnp.sum(h * wo[...], axis=1) + bo[0]
    out[...] = mf * fw[0] + npred * fw[1] + fb[0]


def _make_mlp(interpret=False):
    nb = B // BK
    pack_spec = pl.BlockSpec((BK, PACK), lambda i: (i, 0))
    vec_spec = pl.BlockSpec((BK,), lambda i: (i,))

    def full(shape):
        return pl.BlockSpec(shape, lambda i: tuple(0 for _ in shape))

    smem = pl.BlockSpec(memory_space=pltpu.SMEM)
    return pl.pallas_call(
        _mlp_body,
        grid=(nb,),
        in_specs=[
            pack_spec, pack_spec, vec_spec, vec_spec,
            full((D, 100)), full((D, 100)), full((1, 100)),
            full((100, 50)), full((1, 50)),
            full((50, 20)), full((1, 20)),
            full((1, 20)),
            smem, smem, smem,
        ],
        out_specs=vec_spec,
        out_shape=jax.ShapeDtypeStruct((B,), jnp.float32),
        interpret=interpret,
    )


_mlp = _make_mlp()


def kernel(user_indices, item_indices, mf_user_emb, mf_item_emb,
           mf_user_bias, mf_item_bias, ncf_user_emb, ncf_item_emb,
           W1, b1, W2, b2, W3, b3, W_out, b_out, fusion_W, fusion_b):
    uidx = user_indices.astype(jnp.int32)
    iidx = item_indices.astype(jnp.int32)
    gu, = _build_gather_users()(uidx, mf_user_emb.T, ncf_user_emb.T)
    gi, ub, ib = _build_gather_items()(
        uidx, iidx, mf_item_emb, ncf_item_emb, mf_user_bias, mf_item_bias)
    return _mlp(
        gu, gi, ub, ib,
        W1[:D], W1[D:], b1.reshape(1, -1),
        W2, b2.reshape(1, -1), W3, b3.reshape(1, -1),
        W_out.reshape(1, -1), b_out,
        fusion_W.reshape(-1), fusion_b)
